# speedup vs baseline: 4.7797x; 4.7797x over previous
import functools

import jax
import jax.numpy as jnp
from jax.experimental import pallas as pl
from jax.experimental.pallas import tpu as pltpu

N = 10000
E = 160000
D = 256
H = 8
DH = D // H
FF = 1024


def _mm_kernel(x_ref, w_ref, o_ref):
    o_ref[...] = jnp.dot(x_ref[...], w_ref[...],
                         preferred_element_type=jnp.float32)


def _mm(x, w, block_rows):
    m, k = x.shape
    _, n = w.shape
    grid = (m // block_rows,)
    return pl.pallas_call(
        _mm_kernel,
        grid=grid,
        in_specs=[
            pl.BlockSpec((block_rows, k), lambda i: (i, 0)),
            pl.BlockSpec((k, n), lambda i: (0, 0)),
        ],
        out_specs=pl.BlockSpec((block_rows, n), lambda i: (i, 0)),
        out_shape=jax.ShapeDtypeStruct((m, n), jnp.float32),
    )(x, w)


def _ffn_kernel(x_ref, w1_ref, b1_ref, w2_ref, b2_ref, o_ref):
    h = jnp.dot(x_ref[...], w1_ref[...], preferred_element_type=jnp.float32)
    h = jnp.maximum(h + b1_ref[...], 0.0)
    o_ref[...] = jnp.dot(h, w2_ref[...],
                         preferred_element_type=jnp.float32) + b2_ref[...]


def _ffn(x, w1, b1, w2, b2, block_rows):
    m, k = x.shape
    f = w1.shape[1]
    n = w2.shape[1]
    grid = (m // block_rows,)
    return pl.pallas_call(
        _ffn_kernel,
        grid=grid,
        in_specs=[
            pl.BlockSpec((block_rows, k), lambda i: (i, 0)),
            pl.BlockSpec((k, f), lambda i: (0, 0)),
            pl.BlockSpec((1, f), lambda i: (0, 0)),
            pl.BlockSpec((f, n), lambda i: (0, 0)),
            pl.BlockSpec((1, n), lambda i: (0, 0)),
        ],
        out_specs=pl.BlockSpec((block_rows, n), lambda i: (i, 0)),
        out_shape=jax.ShapeDtypeStruct((m, n), jnp.float32),
    )(x, w1, b1.reshape(1, f), w2, b2.reshape(1, n))


def kernel(nfeats, efeats, edge_index, Wn, bn, Wni, Wnj, Wfij, attn,
           W1, b1, W2, b2, g_pre, b_pre, g_e, b_e):
    src = edge_index[0]
    dst = edge_index[1]

    Wcat = jnp.concatenate([Wni, Wnj, Wn], axis=1)
    proj = _mm(nfeats, Wcat, block_rows=1000)
    f_ni = proj[:, :D]
    f_nj = proj[:, D:2 * D]
    h = proj[:, 2 * D:] + bn

    f_fij = _mm(efeats, Wfij, block_rows=2000)

    f_tmp = f_ni[src] + f_nj[dst] + f_fij
    f_out = jax.nn.leaky_relu(f_tmp, negative_slope=0.2)
    e = jnp.sum(f_out.reshape(E, H, DH) * attn, axis=-1)
    emax = jax.ops.segment_max(e, dst, num_segments=N)
    emax = jnp.where(jnp.isfinite(emax), emax, 0.0)
    ee = jnp.exp(e - emax[dst])
    esum = jax.ops.segment_sum(ee, dst, num_segments=N)
    a = ee / (esum[dst] + 1e-9)
    m = h[src].reshape(E, H, DH) * a[:, :, None]
    h_out = jax.ops.segment_sum(m.reshape(E, D), dst, num_segments=N)

    out_n = h_out + nfeats
    mu = jnp.mean(out_n, axis=0)
    var = jnp.var(out_n, axis=0)
    hff_in = (out_n - mu) / jnp.sqrt(var + 1e-5) * g_pre + b_pre
    hff = _ffn(hff_in, W1, b1, W2, b2, block_rows=1000)
    out_n = out_n + hff

    out_e = f_out + efeats
    mu_e = jnp.mean(out_e, axis=0)
    var_e = jnp.var(out_e, axis=0)
    out_e = (out_e - mu_e) / jnp.sqrt(var_e + 1e-5) * g_e + b_e
    return out_n, out_e


# speedup vs baseline: 5.7399x; 1.2009x over previous
import functools

import jax
import jax.numpy as jnp
from jax import lax
from jax.experimental import pallas as pl
from jax.experimental.pallas import tpu as pltpu
from jax.experimental.pallas import tpu_sc as plsc

N = 10000
E = 160000
D = 256
H = 8
DH = D // H
FF = 1024

_NC = 2
_NS = 16
_NW = _NC * _NS
_EPW = E // _NW
_GB = 128
_NFB = _EPW // _GB
_TAIL = _EPW - _NFB * _GB

_SC_MESH = plsc.VectorSubcoreMesh(
    core_axis_name="c", subcore_axis_name="s", num_cores=_NC, num_subcores=_NS)


def _worker_id():
    return lax.axis_index("s") * _NC + lax.axis_index("c")


def _gather2_body(fni_hbm, fnj_hbm, src_hbm, dst_hbm, gi_out, gj_out,
                  srcv, dstv, gi_v, gj_v, s1, s2):
    base_w = _worker_id() * _EPW

    def do_block(base, b):
        pltpu.sync_copy(src_hbm.at[pl.ds(base, b)], srcv.at[pl.ds(0, b)])
        pltpu.sync_copy(dst_hbm.at[pl.ds(base, b)], dstv.at[pl.ds(0, b)])
        c1 = pltpu.async_copy(fni_hbm.at[srcv.at[pl.ds(0, b)]],
                              gi_v.at[pl.ds(0, b)], s1)
        c2 = pltpu.async_copy(fnj_hbm.at[dstv.at[pl.ds(0, b)]],
                              gj_v.at[pl.ds(0, b)], s2)
        c1.wait()
        c2.wait()
        pltpu.sync_copy(gi_v.at[pl.ds(0, b)], gi_out.at[pl.ds(base, b)])
        pltpu.sync_copy(gj_v.at[pl.ds(0, b)], gj_out.at[pl.ds(base, b)])

    def body(i, _):
        do_block(base_w + i * _GB, _GB)
        return 0

    lax.fori_loop(0, _NFB, body, 0)
    do_block(base_w + _NFB * _GB, _TAIL)


_gather2 = pl.kernel(
    _gather2_body,
    out_type=(jax.ShapeDtypeStruct((E, D), jnp.float32),
              jax.ShapeDtypeStruct((E, D), jnp.float32)),
    mesh=_SC_MESH,
    scratch_types=[
        pltpu.VMEM((_GB,), jnp.int32),
        pltpu.VMEM((_GB,), jnp.int32),
        pltpu.VMEM((_GB, D), jnp.float32),
        pltpu.VMEM((_GB, D), jnp.float32),
        pltpu.SemaphoreType.DMA,
        pltpu.SemaphoreType.DMA,
    ],
)


def _mm_kernel(x_ref, w_ref, o_ref):
    o_ref[...] = jnp.dot(x_ref[...], w_ref[...],
                         preferred_element_type=jnp.float32)


def _mm(x, w, block_rows):
    m, k = x.shape
    _, n = w.shape
    grid = (m // block_rows,)
    return pl.pallas_call(
        _mm_kernel,
        grid=grid,
        in_specs=[
            pl.BlockSpec((block_rows, k), lambda i: (i, 0)),
            pl.BlockSpec((k, n), lambda i: (0, 0)),
        ],
        out_specs=pl.BlockSpec((block_rows, n), lambda i: (i, 0)),
        out_shape=jax.ShapeDtypeStruct((m, n), jnp.float32),
    )(x, w)


def _ffn_kernel(x_ref, w1_ref, b1_ref, w2_ref, b2_ref, o_ref):
    h = jnp.dot(x_ref[...], w1_ref[...], preferred_element_type=jnp.float32)
    h = jnp.maximum(h + b1_ref[...], 0.0)
    o_ref[...] = jnp.dot(h, w2_ref[...],
                         preferred_element_type=jnp.float32) + b2_ref[...]


def _ffn(x, w1, b1, w2, b2, block_rows):
    m, k = x.shape
    f = w1.shape[1]
    n = w2.shape[1]
    grid = (m // block_rows,)
    return pl.pallas_call(
        _ffn_kernel,
        grid=grid,
        in_specs=[
            pl.BlockSpec((block_rows, k), lambda i: (i, 0)),
            pl.BlockSpec((k, f), lambda i: (0, 0)),
            pl.BlockSpec((1, f), lambda i: (0, 0)),
            pl.BlockSpec((f, n), lambda i: (0, 0)),
            pl.BlockSpec((1, n), lambda i: (0, 0)),
        ],
        out_specs=pl.BlockSpec((block_rows, n), lambda i: (i, 0)),
        out_shape=jax.ShapeDtypeStruct((m, n), jnp.float32),
    )(x, w1, b1.reshape(1, f), w2, b2.reshape(1, n))


def kernel(nfeats, efeats, edge_index, Wn, bn, Wni, Wnj, Wfij, attn,
           W1, b1, W2, b2, g_pre, b_pre, g_e, b_e):
    src = edge_index[0]
    dst = edge_index[1]

    Wcat = jnp.concatenate([Wni, Wnj, Wn], axis=1)
    proj = _mm(nfeats, Wcat, block_rows=1000)
    f_ni = proj[:, :D]
    f_nj = proj[:, D:2 * D]
    h = proj[:, 2 * D:] + bn

    f_fij = _mm(efeats, Wfij, block_rows=2000)

    gi, gj = _gather2(f_ni, f_nj, src, dst)

    f_tmp = gi + gj + f_fij
    f_out = jax.nn.leaky_relu(f_tmp, negative_slope=0.2)
    e = jnp.sum(f_out.reshape(E, H, DH) * attn, axis=-1)
    emax = jax.ops.segment_max(e, dst, num_segments=N)
    emax = jnp.where(jnp.isfinite(emax), emax, 0.0)
    ee = jnp.exp(e - emax[dst])
    esum = jax.ops.segment_sum(ee, dst, num_segments=N)
    a = ee / (esum[dst] + 1e-9)
    m = h[src].reshape(E, H, DH) * a[:, :, None]
    h_out = jax.ops.segment_sum(m.reshape(E, D), dst, num_segments=N)

    out_n = h_out + nfeats
    mu = jnp.mean(out_n, axis=0)
    var = jnp.var(out_n, axis=0)
    hff_in = (out_n - mu) / jnp.sqrt(var + 1e-5) * g_pre + b_pre
    hff = _ffn(hff_in, W1, b1, W2, b2, block_rows=1000)
    out_n = out_n + hff

    out_e = f_out + efeats
    mu_e = jnp.mean(out_e, axis=0)
    var_e = jnp.var(out_e, axis=0)
    out_e = (out_e - mu_e) / jnp.sqrt(var_e + 1e-5) * g_e + b_e
    return out_n, out_e


# speedup vs baseline: 13.3345x; 2.3231x over previous
import functools

import jax
import jax.numpy as jnp
from jax import lax
from jax.experimental import pallas as pl
from jax.experimental.pallas import tpu as pltpu
from jax.experimental.pallas import tpu_sc as plsc

N = 10000
E = 160000
D = 256
H = 8
DH = D // H
FF = 1024

_NC = 2
_NS = 16
_NW = _NC * _NS
_EPW = E // _NW
_GB = 128
_NFB = _EPW // _GB
_TAIL = _EPW - _NFB * _GB

_SC_MESH = plsc.VectorSubcoreMesh(
    core_axis_name="c", subcore_axis_name="s", num_cores=_NC, num_subcores=_NS)


def _worker_id():
    return lax.axis_index("s") * _NC + lax.axis_index("c")


def _gather2_body(fni_hbm, fnj_hbm, src_hbm, dst_hbm, gi_out, gj_out,
                  srcv, dstv, gi_v, gj_v, s1, s2):
    base_w = _worker_id() * _EPW

    def do_block(base, b):
        pltpu.sync_copy(src_hbm.at[pl.ds(base, b)], srcv.at[pl.ds(0, b)])
        pltpu.sync_copy(dst_hbm.at[pl.ds(base, b)], dstv.at[pl.ds(0, b)])
        c1 = pltpu.async_copy(fni_hbm.at[srcv.at[pl.ds(0, b)]],
                              gi_v.at[pl.ds(0, b)], s1)
        c2 = pltpu.async_copy(fnj_hbm.at[dstv.at[pl.ds(0, b)]],
                              gj_v.at[pl.ds(0, b)], s2)
        c1.wait()
        c2.wait()
        pltpu.sync_copy(gi_v.at[pl.ds(0, b)], gi_out.at[pl.ds(base, b)])
        pltpu.sync_copy(gj_v.at[pl.ds(0, b)], gj_out.at[pl.ds(base, b)])

    def body(i, _):
        do_block(base_w + i * _GB, _GB)
        return 0

    lax.fori_loop(0, _NFB, body, 0)
    do_block(base_w + _NFB * _GB, _TAIL)


_gather2 = pl.kernel(
    _gather2_body,
    out_type=(jax.ShapeDtypeStruct((E, D), jnp.float32),
              jax.ShapeDtypeStruct((E, D), jnp.float32)),
    mesh=_SC_MESH,
    scratch_types=[
        pltpu.VMEM((_GB,), jnp.int32),
        pltpu.VMEM((_GB,), jnp.int32),
        pltpu.VMEM((_GB, D), jnp.float32),
        pltpu.VMEM((_GB, D), jnp.float32),
        pltpu.SemaphoreType.DMA,
        pltpu.SemaphoreType.DMA,
    ],
)


def _mm_kernel(x_ref, w_ref, o_ref):
    o_ref[...] = jnp.dot(x_ref[...], w_ref[...],
                         preferred_element_type=jnp.float32)


def _mm(x, w, block_rows):
    m, k = x.shape
    _, n = w.shape
    grid = (m // block_rows,)
    return pl.pallas_call(
        _mm_kernel,
        grid=grid,
        in_specs=[
            pl.BlockSpec((block_rows, k), lambda i: (i, 0)),
            pl.BlockSpec((k, n), lambda i: (0, 0)),
        ],
        out_specs=pl.BlockSpec((block_rows, n), lambda i: (i, 0)),
        out_shape=jax.ShapeDtypeStruct((m, n), jnp.float32),
    )(x, w)


def _ffn_kernel(x_ref, w1_ref, b1_ref, w2_ref, b2_ref, o_ref):
    h = jnp.dot(x_ref[...], w1_ref[...], preferred_element_type=jnp.float32)
    h = jnp.maximum(h + b1_ref[...], 0.0)
    o_ref[...] = jnp.dot(h, w2_ref[...],
                         preferred_element_type=jnp.float32) + b2_ref[...]


def _ffn(x, w1, b1, w2, b2, block_rows):
    m, k = x.shape
    f = w1.shape[1]
    n = w2.shape[1]
    grid = (m // block_rows,)
    return pl.pallas_call(
        _ffn_kernel,
        grid=grid,
        in_specs=[
            pl.BlockSpec((block_rows, k), lambda i: (i, 0)),
            pl.BlockSpec((k, f), lambda i: (0, 0)),
            pl.BlockSpec((1, f), lambda i: (0, 0)),
            pl.BlockSpec((f, n), lambda i: (0, 0)),
            pl.BlockSpec((1, n), lambda i: (0, 0)),
        ],
        out_specs=pl.BlockSpec((block_rows, n), lambda i: (i, 0)),
        out_shape=jax.ShapeDtypeStruct((m, n), jnp.float32),
    )(x, w1, b1.reshape(1, f), w2, b2.reshape(1, n))


_NPAD = 10112
_NPW = _NPAD // _NS


def _msg_body(src_hbm, dst_hbm, ee_hbm, h0_hbm, h1_hbm, zeros_hbm,
              out, srcv, dstv, eev, hv, srcv8, dstv8, eev8, hv8,
              s1, s3, acc):
    cid = lax.axis_index("c")
    sid = lax.axis_index("s")
    base_w = _worker_id() * _EPW

    for p in (0, 1, 2):
        pltpu.sync_copy(zeros_hbm, acc.at[pl.ds(sid * _NPW, _NPW)])
        if p == 2:
            def zrow(i, _):
                for v in range(8):
                    hv[i, pl.ds(16 * v, 16)] = jnp.zeros((16,), jnp.float32)
                return 0
            lax.fori_loop(0, _GB, zrow, 0)
            for i in range(_TAIL):
                for v in range(8):
                    hv8[i, pl.ds(16 * v, 16)] = jnp.zeros((16,), jnp.float32)
        plsc.subcore_barrier()

        def do_block(base, sv, dv, ev, hvb, b, p=p):
            pltpu.sync_copy(dst_hbm.at[pl.ds(base, b)], dv)
            c1 = pltpu.async_copy(ee_hbm.at[pl.ds(base, b)], ev, s1)
            if p < 2:
                pltpu.sync_copy(src_hbm.at[pl.ds(base, b)], sv)
                h_hbm = h0_hbm if p == 0 else h1_hbm
                c3 = pltpu.async_copy(h_hbm.at[sv], hvb, s3)
                c1.wait()
                c3.wait()

                def edge(i, _):
                    av = ev[i, :]
                    for v in range(8):
                        s = jnp.broadcast_to(av[4 * p + v // 2], (16,))
                        hvb[i, pl.ds(16 * v, 16)] = (
                            hvb[i, pl.ds(16 * v, 16)] * s)
                    return 0
            else:
                c1.wait()

                def edge(i, _):
                    hvb[i, pl.ds(0, 16)] = ev[i, :]
                    return 0

            lax.fori_loop(0, b, edge, 0)
            pltpu.sync_copy(hvb, acc.at[dv], add=True)

        def body(i, _, do_block=do_block):
            do_block(base_w + i * _GB, srcv, dstv, eev, hv, _GB)
            return 0

        lax.fori_loop(0, _NFB, body, 0)
        do_block(base_w + _NFB * _GB, srcv8, dstv8, eev8, hv8, _TAIL)

        plsc.subcore_barrier()
        pltpu.sync_copy(acc.at[pl.ds(sid * _NPW, _NPW)],
                        out.at[cid, p, pl.ds(sid * _NPW, _NPW)])
        plsc.subcore_barrier()


_msg_scatter = pl.kernel(
    _msg_body,
    out_type=jax.ShapeDtypeStruct((_NC, 3, _NPAD, 128), jnp.float32),
    mesh=_SC_MESH,
    scratch_types=[
        pltpu.VMEM((_GB,), jnp.int32),
        pltpu.VMEM((_GB,), jnp.int32),
        pltpu.VMEM((_GB, 16), jnp.float32),
        pltpu.VMEM((_GB, 128), jnp.float32),
        pltpu.VMEM((_TAIL,), jnp.int32),
        pltpu.VMEM((_TAIL,), jnp.int32),
        pltpu.VMEM((_TAIL, 16), jnp.float32),
        pltpu.VMEM((_TAIL, 128), jnp.float32),
        pltpu.SemaphoreType.DMA,
        pltpu.SemaphoreType.DMA,
        pltpu.VMEM_SHARED((_NPAD, 128), jnp.float32),
    ],
)


def kernel(nfeats, efeats, edge_index, Wn, bn, Wni, Wnj, Wfij, attn,
           W1, b1, W2, b2, g_pre, b_pre, g_e, b_e):
    src = edge_index[0]
    dst = edge_index[1]

    Wcat = jnp.concatenate([Wni, Wnj, Wn], axis=1)
    proj = _mm(nfeats, Wcat, block_rows=1000)
    f_ni = proj[:, :D]
    f_nj = proj[:, D:2 * D]
    h = proj[:, 2 * D:] + bn

    f_fij = _mm(efeats, Wfij, block_rows=2000)

    gi, gj = _gather2(f_ni, f_nj, src, dst)

    f_tmp = gi + gj + f_fij
    f_out = jax.nn.leaky_relu(f_tmp, negative_slope=0.2)
    e = jnp.sum(f_out.reshape(E, H, DH) * attn, axis=-1)
    g = jnp.max(e)
    ee = jnp.exp(e - g)
    ee16 = jnp.concatenate([ee, jnp.zeros((E, 8), jnp.float32)], axis=1)

    h0 = h[:, :128]
    h1 = h[:, 128:]
    hp = _msg_scatter(src, dst, ee16, h0, h1,
                      jnp.zeros((_NPW, 128), jnp.float32))
    h_raw = jnp.concatenate([hp[0, 0, :N] + hp[1, 0, :N],
                             hp[0, 1, :N] + hp[1, 1, :N]], axis=1)
    esum16 = hp[0, 2, :N, :16] + hp[1, 2, :N, :16]
    sel = jnp.repeat(jnp.eye(16, dtype=jnp.float32)[:, :H], DH, axis=1)
    h_out = h_raw / (esum16 @ sel + 1e-9)

    out_n = h_out + nfeats
    mu = jnp.mean(out_n, axis=0)
    var = jnp.var(out_n, axis=0)
    hff_in = (out_n - mu) / jnp.sqrt(var + 1e-5) * g_pre + b_pre
    hff = _ffn(hff_in, W1, b1, W2, b2, block_rows=1000)
    out_n = out_n + hff

    out_e = f_out + efeats
    mu_e = jnp.mean(out_e, axis=0)
    var_e = jnp.var(out_e, axis=0)
    out_e = (out_e - mu_e) / jnp.sqrt(var_e + 1e-5) * g_e + b_e
    return out_n, out_e
